# baseline (speedup 1.0000x reference)
import jax
import jax.numpy as jnp
from jax import lax
from jax.experimental import pallas as pl
from jax.experimental.pallas import tpu as pltpu
from jax.experimental.pallas import tpu_sc as plsc

NFEAT = 128
HID = 128
N = 10000
E = 320000
C = 128
NW = 32
CPT = 160
NCHUNK = NW * CPT
EP = NCHUNK * C
CM = 64
CPTM = EP // (NW * CM)
NCHUNKM = NW * CPTM
NACC = 10112
DUMMY = N
RPT = NACC // 16

MB = 1000
GRID_N = N // MB


def _sc_mesh():
    return plsc.VectorSubcoreMesh(core_axis_name="c", subcore_axis_name="s",
                                  num_cores=2, num_subcores=16)


def _msg_body(ipk, p_hbm, q_hbm, r_hbm, hacc_hbm,
              ibuf, buf, bufp, bufq, hsh, sem):
    cid = lax.axis_index("c")
    sid = lax.axis_index("s")
    wid = cid * 16 + sid
    base = sid * RPT

    zero16 = jnp.zeros((16,), jnp.float32)

    def zrow(i, carry):
        for g in range(HID // 16):
            bufp[i, pl.ds(g * 16, 16)] = zero16
        return carry

    lax.fori_loop(0, CM, zrow, 0)
    for k in range(RPT // CM):
        pltpu.sync_copy(bufp, hsh.at[pl.ds(base + k * CM, CM)])
    rem = RPT % CM
    if rem:
        pltpu.sync_copy(bufp.at[pl.ds(0, rem)],
                        hsh.at[pl.ds(base + (RPT // CM) * CM, rem)])
    plsc.subcore_barrier()

    def chunk(c, carry):
        cidx = wid * CPTM + c
        pltpu.sync_copy(ipk.at[cidx], ibuf)
        rbase = lax.rem(cidx * CM, E)
        d0 = pltpu.async_copy(r_hbm.at[pl.ds(rbase, CM)], buf, sem)
        d1 = pltpu.async_copy(p_hbm.at[ibuf.at[0]], bufp, sem)
        d2 = pltpu.async_copy(q_hbm.at[ibuf.at[1]], bufq, sem)
        d0.wait()
        d1.wait()
        d2.wait()

        def relu_row(i, rc):
            for g in range(HID // 16):
                s = pl.ds(g * 16, 16)
                buf[i, s] = jnp.maximum(buf[i, s] + bufp[i, s] + bufq[i, s],
                                        0.0)
            return rc

        lax.fori_loop(0, CM, relu_row, 0)
        pltpu.sync_copy(buf, hsh.at[ibuf.at[3]], add=True)
        return carry

    lax.fori_loop(0, CPTM, chunk, 0)
    plsc.subcore_barrier()
    pltpu.sync_copy(hsh.at[pl.ds(base, RPT)], hacc_hbm.at[cid, pl.ds(base, RPT)])


def _zero_rows(src, dst, base, n, width_rows):
    for k in range(n // width_rows):
        pltpu.sync_copy(src, dst.at[pl.ds(base + k * width_rows, width_rows)])
    rem = n % width_rows
    if rem:
        pltpu.sync_copy(src.at[pl.ds(0, rem)],
                        dst.at[pl.ds(base + (n // width_rows) * width_rows, rem)])


def _prop_body(ipk4, tab_hbm, acc_hbm, ib0, buf0, zbuf, hsh, gs0):
    cid = lax.axis_index("c")
    sid = lax.axis_index("s")
    wid = cid * 16 + sid
    base = sid * RPT

    zero16 = jnp.zeros((16,), jnp.float32)

    def zrow(i, carry):
        for g in range(HID // 16):
            zbuf[i, pl.ds(g * 16, 16)] = zero16
        return carry

    lax.fori_loop(0, C, zrow, 0)
    _zero_rows(zbuf, hsh, base, RPT, C)
    plsc.subcore_barrier()

    cbase = wid * CPT

    def chunk(c, carry):
        pltpu.sync_copy(ipk4.at[cbase + c], ib0)
        pltpu.async_copy(tab_hbm.at[ib0.at[1]], buf0, gs0).wait()
        pltpu.sync_copy(buf0, hsh.at[ib0.at[3]], add=True)
        return carry

    lax.fori_loop(0, CPT, chunk, 0)
    plsc.subcore_barrier()
    pltpu.sync_copy(hsh.at[pl.ds(base, RPT)], acc_hbm.at[cid, pl.ds(base, RPT)])


def _degs_body(ipk2, deg_hbm, ib0, ib1, ib2, ib3, obuf, dsh,
               is0, is1, is2, is3):
    cid = lax.axis_index("c")
    sid = lax.axis_index("s")
    wid = cid * 16 + sid
    base = sid * RPT
    ibs = (ib0, ib1, ib2, ib3)
    isems = (is0, is1, is2, is3)

    zero16 = jnp.zeros((16,), jnp.float32)
    ones16 = jnp.ones((16,), jnp.float32)

    def zrow(i, carry):
        for g in range(HID // 16):
            obuf[i, pl.ds(g * 16, 16)] = zero16
        return carry

    lax.fori_loop(0, C, zrow, 0)
    _zero_rows(obuf, dsh, base, RPT, C)

    def orow(i, carry):
        for g in range(HID // 16):
            obuf[i, pl.ds(g * 16, 16)] = ones16
        return carry

    lax.fori_loop(0, C, orow, 0)
    plsc.subcore_barrier()

    cbase = wid * CPT
    for j in range(4):
        pltpu.async_copy(ipk2.at[cbase + j], ibs[j], isems[j])
    pltpu.make_async_copy(ipk2.at[cbase], ibs[0], isems[0]).wait()

    def quad(t, carry):
        c0 = t * 4
        for j in range(4):
            c = c0 + j
            ibc = ibs[j]

            pltpu.sync_copy(obuf, dsh.at[ibc.at[3]], add=True)

            @pl.when(c + 4 < CPT)
            def _():
                pltpu.async_copy(ipk2.at[cbase + c + 4], ibs[j], isems[j])

            @pl.when(c + 1 < CPT)
            def _():
                pltpu.make_async_copy(ipk2.at[cbase + c + 1],
                                      ibs[(j + 1) % 4],
                                      isems[(j + 1) % 4]).wait()
        return carry

    lax.fori_loop(0, CPT // 4, quad, 0)
    plsc.subcore_barrier()
    pltpu.sync_copy(dsh.at[pl.ds(base, RPT)], deg_hbm.at[cid, pl.ds(base, RPT)])


def _msg_call(ipk, P, Q, R):
    return pl.kernel(
        _msg_body,
        out_type=jax.ShapeDtypeStruct((2, NACC, HID), jnp.float32),
        mesh=_sc_mesh(),
        scratch_types=[
            pltpu.VMEM((4, CM), jnp.int32),
            pltpu.VMEM((CM, HID), jnp.float32),
            pltpu.VMEM((CM, HID), jnp.float32),
            pltpu.VMEM((CM, HID), jnp.float32),
            pltpu.VMEM_SHARED((NACC, HID), jnp.float32),
            pltpu.SemaphoreType.DMA,
        ],
    )(ipk, P, Q, R)


def _prop_call(ipk2, table):
    return pl.kernel(
        _prop_body,
        out_type=jax.ShapeDtypeStruct((2, NACC, HID), jnp.float32),
        mesh=_sc_mesh(),
        scratch_types=[
            pltpu.VMEM((4, C), jnp.int32),
            pltpu.VMEM((C, HID), jnp.float32),
            pltpu.VMEM((C, HID), jnp.float32),
            pltpu.VMEM_SHARED((NACC, HID), jnp.float32),
            pltpu.SemaphoreType.DMA,
        ],
    )(ipk2, table)


def _degs_call(ipk2):
    return pl.kernel(
        _degs_body,
        out_type=jax.ShapeDtypeStruct((2, NACC, HID), jnp.float32),
        mesh=_sc_mesh(),
        scratch_types=[
            pltpu.VMEM((4, C), jnp.int32),
            pltpu.VMEM((4, C), jnp.int32),
            pltpu.VMEM((4, C), jnp.int32),
            pltpu.VMEM((4, C), jnp.int32),
            pltpu.VMEM((C, HID), jnp.float32),
            pltpu.VMEM_SHARED((NACC, HID), jnp.float32),
            pltpu.SemaphoreType.DMA,
            pltpu.SemaphoreType.DMA,
            pltpu.SemaphoreType.DMA,
            pltpu.SemaphoreType.DMA,
        ],
    )(ipk2)


def _pre1_body(xf_ref, w_ref, o_ref):
    o_ref[...] = jnp.dot(xf_ref[...], w_ref[...],
                         preferred_element_type=jnp.float32)


def _pre1(xf, w01):
    return pl.pallas_call(
        _pre1_body,
        grid=(GRID_N,),
        in_specs=[pl.BlockSpec((MB, NFEAT), lambda i: (i, 0)),
                  pl.BlockSpec((NFEAT, 2 * HID), lambda i: (0, 0))],
        out_specs=pl.BlockSpec((MB, 2 * HID), lambda i: (i, 0)),
        out_shape=jax.ShapeDtypeStruct((N, 2 * HID), jnp.float32),
    )(xf, w01)


def _pre2_body(ea_ref, w_ref, b_ref, r_ref):
    r_ref[...] = (jnp.dot(ea_ref[...], w_ref[...],
                          preferred_element_type=jnp.float32) + b_ref[...])


def _pre2(ea, w2, b1):
    EB = 8000
    return pl.pallas_call(
        _pre2_body,
        grid=(E // EB,),
        in_specs=[pl.BlockSpec((EB, 16), lambda i: (i, 0)),
                  pl.BlockSpec((16, HID), lambda i: (0, 0)),
                  pl.BlockSpec((1, HID), lambda i: (0, 0))],
        out_specs=pl.BlockSpec((EB, HID), lambda i: (i, 0)),
        out_shape=jax.ShapeDtypeStruct((E, HID), jnp.float32),
    )(ea, w2, b1)


def _combine_body(hacc_ref, deg_ref, w2_ref, b2_ref, h_ref, t1_ref, dist_ref):
    hs = hacc_ref[0] + hacc_ref[1]
    deg = (deg_ref[0, :, :1] + deg_ref[1, :, :1])
    h = jnp.dot(hs, w2_ref[...], preferred_element_type=jnp.float32) \
        + deg * b2_ref[...]
    dist = jnp.where(deg > 0, lax.rsqrt(deg), 0.0)
    h_ref[...] = h
    t1_ref[...] = dist * h
    dist_ref[...] = dist


def _combine(hacc, degp, w2, b2):
    return pl.pallas_call(
        _combine_body,
        grid=(GRID_N,),
        in_specs=[pl.BlockSpec((2, MB, HID), lambda i: (0, i, 0)),
                  pl.BlockSpec((2, MB, HID), lambda i: (0, i, 0)),
                  pl.BlockSpec((HID, HID), lambda i: (0, 0)),
                  pl.BlockSpec((1, HID), lambda i: (0, 0))],
        out_specs=[pl.BlockSpec((MB, HID), lambda i: (i, 0)),
                   pl.BlockSpec((MB, HID), lambda i: (i, 0)),
                   pl.BlockSpec((MB, 1), lambda i: (i, 0))],
        out_shape=[jax.ShapeDtypeStruct((N, HID), jnp.float32),
                   jax.ShapeDtypeStruct((N, HID), jnp.float32),
                   jax.ShapeDtypeStruct((N, 1), jnp.float32)],
    )(hacc, degp, w2, b2)


def _mid_body(u_ref, dist_ref, su_ref, t2_ref):
    u = u_ref[0] + u_ref[1]
    dist = dist_ref[...]
    su = dist * u
    su_ref[...] = su
    t2_ref[...] = dist * su


def _mid(uacc, dist):
    return pl.pallas_call(
        _mid_body,
        grid=(GRID_N,),
        in_specs=[pl.BlockSpec((2, MB, HID), lambda i: (0, i, 0)),
                  pl.BlockSpec((MB, 1), lambda i: (i, 0))],
        out_specs=[pl.BlockSpec((MB, HID), lambda i: (i, 0)),
                   pl.BlockSpec((MB, HID), lambda i: (i, 0))],
        out_shape=[jax.ShapeDtypeStruct((N, HID), jnp.float32),
                   jax.ShapeDtypeStruct((N, HID), jnp.float32)],
    )(uacc, dist)


def _post_relu_body(h_ref, su_ref, v_ref, dist_ref, w_ref, b_ref,
                    hn_ref, tn_ref):
    dist = dist_ref[...]
    sv = dist * (v_ref[0] + v_ref[1])
    out = (jnp.dot(h_ref[...], w_ref[0], preferred_element_type=jnp.float32)
           + jnp.dot(su_ref[...], w_ref[1], preferred_element_type=jnp.float32)
           + jnp.dot(sv, w_ref[2], preferred_element_type=jnp.float32)
           + b_ref[...])
    hn = jnp.maximum(out, 0.0)
    hn_ref[...] = hn
    tn_ref[...] = dist * hn


def _post_final_body(h_ref, su_ref, v_ref, dist_ref, w_ref, b_ref, out_ref):
    dist = dist_ref[...]
    sv = dist * (v_ref[0] + v_ref[1])
    out_ref[...] = (
        jnp.dot(h_ref[...], w_ref[0], preferred_element_type=jnp.float32)
        + jnp.dot(su_ref[...], w_ref[1], preferred_element_type=jnp.float32)
        + jnp.dot(sv, w_ref[2], preferred_element_type=jnp.float32)
        + b_ref[...])


def _post(h, su, vacc, dist, w, b, final):
    in_specs = [pl.BlockSpec((MB, HID), lambda i: (i, 0)),
                pl.BlockSpec((MB, HID), lambda i: (i, 0)),
                pl.BlockSpec((2, MB, HID), lambda i: (0, i, 0)),
                pl.BlockSpec((MB, 1), lambda i: (i, 0)),
                pl.BlockSpec((3, HID, HID), lambda i: (0, 0, 0)),
                pl.BlockSpec((1, HID), lambda i: (0, 0))]
    if final:
        return pl.pallas_call(
            _post_final_body,
            grid=(GRID_N,),
            in_specs=in_specs,
            out_specs=pl.BlockSpec((MB, HID), lambda i: (i, 0)),
            out_shape=jax.ShapeDtypeStruct((N, HID), jnp.float32),
        )(h, su, vacc, dist, w, b)
    return pl.pallas_call(
        _post_relu_body,
        grid=(GRID_N,),
        in_specs=in_specs,
        out_specs=[pl.BlockSpec((MB, HID), lambda i: (i, 0)),
                   pl.BlockSpec((MB, HID), lambda i: (i, 0))],
        out_shape=[jax.ShapeDtypeStruct((N, HID), jnp.float32),
                   jax.ShapeDtypeStruct((N, HID), jnp.float32)],
    )(h, su, vacc, dist, w, b)


def kernel(x, edge_index, edge_attr, ea_w1, ea_b1, ea_w2, ea_b2,
           tag1_w, tag1_b, tag2_w, tag2_b, tag3_w, tag3_b):
    xf = x[:, 4:4 + NFEAT]
    ei = edge_index.astype(jnp.int32)
    row0, col0 = ei[0], ei[1]
    directed = ~jnp.any((row0 == col0[0]) & (col0 == row0[0]))

    pad = EP - 2 * E
    zpad = jnp.zeros((pad,), jnp.int32)
    ar = jnp.arange(E, dtype=jnp.int32)
    gcol = jnp.concatenate([col0, row0, zpad])
    grow = jnp.concatenate([row0, col0, zpad])
    ridx = jnp.concatenate([ar, ar, zpad])
    dummy_rev = DUMMY + (ar % (NACC - N))
    dummy_pad = DUMMY + (jnp.arange(pad, dtype=jnp.int32) % (NACC - N))
    scat = jnp.concatenate([col0, jnp.where(directed, row0, dummy_rev),
                            dummy_pad])
    packed = jnp.stack([gcol, grow, ridx, scat], axis=0)
    ipkm = packed.reshape(4, NCHUNKM, CM).transpose(1, 0, 2)
    ipk2 = packed.reshape(4, NCHUNK, C).transpose(1, 0, 2)

    w01 = jnp.concatenate([ea_w1[:NFEAT], ea_w1[NFEAT:2 * NFEAT]], axis=1)
    PQ = _pre1(xf, w01)
    P = PQ[:, :HID]
    Q = PQ[:, HID:]
    R = _pre2(edge_attr, ea_w1[2 * NFEAT:], ea_b1.reshape(1, HID))

    hacc = _msg_call(ipkm, P, Q, R)
    degp = _degs_call(ipk2)
    h, t, dist = _combine(hacc, degp, ea_w2, ea_b2.reshape(1, HID))

    for (w, b, final) in ((tag1_w, tag1_b, False),
                          (tag2_w, tag2_b, False),
                          (tag3_w, tag3_b, True)):
        uacc = _prop_call(ipk2, t)
        su, t2 = _mid(uacc, dist)
        vacc = _prop_call(ipk2, t2)
        res = _post(h, su, vacc, dist, w, b.reshape(1, HID), final)
        if final:
            return res
        h, t = res


# speedup vs baseline: 1.9927x; 1.9927x over previous
import jax
import jax.numpy as jnp
from jax import lax
from jax.experimental import pallas as pl
from jax.experimental.pallas import tpu as pltpu
from jax.experimental.pallas import tpu_sc as plsc

NFEAT = 128
HID = 128
N = 10000
E = 320000
C = 128
NW = 32
CPT = 157
NCHUNK = NW * CPT
EP = NCHUNK * C
CM = 64
CPTM = EP // (NW * CM)
NCHUNKM = NW * CPTM
NACC = 10112
DUMMY = N
RPT = NACC // 16

MB = 1000
GRID_N = N // MB


def _sc_mesh():
    return plsc.VectorSubcoreMesh(core_axis_name="c", subcore_axis_name="s",
                                  num_cores=2, num_subcores=16)


def _msg_body(ipk, p_hbm, q_hbm, r_hbm, hacc_hbm,
              ibuf, buf, bufp, bufq, hsh, sem):
    cid = lax.axis_index("c")
    sid = lax.axis_index("s")
    wid = cid * 16 + sid
    base = sid * RPT

    zero16 = jnp.zeros((16,), jnp.float32)

    def zrow(i, carry):
        for g in range(HID // 16):
            bufp[i, pl.ds(g * 16, 16)] = zero16
        return carry

    lax.fori_loop(0, CM, zrow, 0)
    for k in range(RPT // CM):
        pltpu.sync_copy(bufp, hsh.at[pl.ds(base + k * CM, CM)])
    rem = RPT % CM
    if rem:
        pltpu.sync_copy(bufp.at[pl.ds(0, rem)],
                        hsh.at[pl.ds(base + (RPT // CM) * CM, rem)])
    plsc.subcore_barrier()

    def chunk(c, carry):
        cidx = wid * CPTM + c
        pltpu.sync_copy(ipk.at[cidx], ibuf)
        rbase = lax.rem(cidx * CM, E)
        d0 = pltpu.async_copy(r_hbm.at[pl.ds(rbase, CM)], buf, sem)
        d1 = pltpu.async_copy(p_hbm.at[ibuf.at[0]], bufp, sem)
        d2 = pltpu.async_copy(q_hbm.at[ibuf.at[1]], bufq, sem)
        d0.wait()
        d1.wait()
        d2.wait()

        def relu_row(i, rc):
            for g in range(HID // 16):
                s = pl.ds(g * 16, 16)
                buf[i, s] = jnp.maximum(buf[i, s] + bufp[i, s] + bufq[i, s],
                                        0.0)
            return rc

        lax.fori_loop(0, CM, relu_row, 0)
        pltpu.sync_copy(buf, hsh.at[ibuf.at[3]], add=True)
        return carry

    lax.fori_loop(0, CPTM, chunk, 0)
    plsc.subcore_barrier()
    pltpu.sync_copy(hsh.at[pl.ds(base, RPT)], hacc_hbm.at[cid, pl.ds(base, RPT)])


def _zero_rows(src, dst, base, n, width_rows):
    for k in range(n // width_rows):
        pltpu.sync_copy(src, dst.at[pl.ds(base + k * width_rows, width_rows)])
    rem = n % width_rows
    if rem:
        pltpu.sync_copy(src.at[pl.ds(0, rem)],
                        dst.at[pl.ds(base + (n // width_rows) * width_rows, rem)])


def _prop_body(ipk4, tab_hbm, acc_hbm, ib0, buf0, zbuf, hsh, gs0):
    cid = lax.axis_index("c")
    sid = lax.axis_index("s")
    wid = cid * 16 + sid
    base = sid * RPT

    zero16 = jnp.zeros((16,), jnp.float32)

    def zrow(i, carry):
        for g in range(HID // 16):
            zbuf[i, pl.ds(g * 16, 16)] = zero16
        return carry

    lax.fori_loop(0, C, zrow, 0)
    _zero_rows(zbuf, hsh, base, RPT, C)
    plsc.subcore_barrier()

    cbase = wid * CPT

    def chunk(c, carry):
        pltpu.sync_copy(ipk4.at[cbase + c], ib0)
        pltpu.async_copy(tab_hbm.at[ib0.at[1]], buf0, gs0).wait()
        pltpu.sync_copy(buf0, hsh.at[ib0.at[3]], add=True)
        return carry

    lax.fori_loop(0, CPT, chunk, 0)
    plsc.subcore_barrier()
    pltpu.sync_copy(hsh.at[pl.ds(base, RPT)], acc_hbm.at[cid, pl.ds(base, RPT)])


def _degs_body(ipk2, deg_hbm, ib0, ib1, ib2, ib3, obuf, dsh,
               is0, is1, is2, is3):
    cid = lax.axis_index("c")
    sid = lax.axis_index("s")
    wid = cid * 16 + sid
    base = sid * RPT
    ibs = (ib0, ib1, ib2, ib3)
    isems = (is0, is1, is2, is3)

    zero16 = jnp.zeros((16,), jnp.float32)
    ones16 = jnp.ones((16,), jnp.float32)

    def zrow(i, carry):
        for g in range(HID // 16):
            obuf[i, pl.ds(g * 16, 16)] = zero16
        return carry

    lax.fori_loop(0, C, zrow, 0)
    _zero_rows(obuf, dsh, base, RPT, C)

    def orow(i, carry):
        for g in range(HID // 16):
            obuf[i, pl.ds(g * 16, 16)] = ones16
        return carry

    lax.fori_loop(0, C, orow, 0)
    plsc.subcore_barrier()

    cbase = wid * CPT

    def chunk(c, carry):
        pltpu.sync_copy(ipk2.at[cbase + c], ibs[0])
        pltpu.sync_copy(obuf, dsh.at[ibs[0].at[3]], add=True)
        return carry

    lax.fori_loop(0, CPT, chunk, 0)
    plsc.subcore_barrier()
    pltpu.sync_copy(dsh.at[pl.ds(base, RPT)], deg_hbm.at[cid, pl.ds(base, RPT)])


def _msg_call(ipk, P, Q, R):
    return pl.kernel(
        _msg_body,
        out_type=jax.ShapeDtypeStruct((2, NACC, HID), jnp.float32),
        mesh=_sc_mesh(),
        scratch_types=[
            pltpu.VMEM((4, CM), jnp.int32),
            pltpu.VMEM((CM, HID), jnp.float32),
            pltpu.VMEM((CM, HID), jnp.float32),
            pltpu.VMEM((CM, HID), jnp.float32),
            pltpu.VMEM_SHARED((NACC, HID), jnp.float32),
            pltpu.SemaphoreType.DMA,
        ],
    )(ipk, P, Q, R)


def _prop_call(ipk2, table):
    return pl.kernel(
        _prop_body,
        out_type=jax.ShapeDtypeStruct((2, NACC, HID), jnp.float32),
        mesh=_sc_mesh(),
        scratch_types=[
            pltpu.VMEM((4, C), jnp.int32),
            pltpu.VMEM((C, HID), jnp.float32),
            pltpu.VMEM((C, HID), jnp.float32),
            pltpu.VMEM_SHARED((NACC, HID), jnp.float32),
            pltpu.SemaphoreType.DMA,
        ],
    )(ipk2, table)


def _degs_call(ipk2):
    return pl.kernel(
        _degs_body,
        out_type=jax.ShapeDtypeStruct((2, NACC, HID), jnp.float32),
        mesh=_sc_mesh(),
        scratch_types=[
            pltpu.VMEM((4, C), jnp.int32),
            pltpu.VMEM((4, C), jnp.int32),
            pltpu.VMEM((4, C), jnp.int32),
            pltpu.VMEM((4, C), jnp.int32),
            pltpu.VMEM((C, HID), jnp.float32),
            pltpu.VMEM_SHARED((NACC, HID), jnp.float32),
            pltpu.SemaphoreType.DMA,
            pltpu.SemaphoreType.DMA,
            pltpu.SemaphoreType.DMA,
            pltpu.SemaphoreType.DMA,
        ],
    )(ipk2)


def _pre1_body(xf_ref, w_ref, o_ref):
    o_ref[...] = jnp.dot(xf_ref[...], w_ref[...],
                         preferred_element_type=jnp.float32)


def _pre1(xf, w01):
    return pl.pallas_call(
        _pre1_body,
        grid=(GRID_N,),
        in_specs=[pl.BlockSpec((MB, NFEAT), lambda i: (i, 0)),
                  pl.BlockSpec((NFEAT, 2 * HID), lambda i: (0, 0))],
        out_specs=pl.BlockSpec((MB, 2 * HID), lambda i: (i, 0)),
        out_shape=jax.ShapeDtypeStruct((N, 2 * HID), jnp.float32),
    )(xf, w01)


def _pre2_body(ea_ref, w_ref, b_ref, r_ref):
    r_ref[...] = (jnp.dot(ea_ref[...], w_ref[...],
                          preferred_element_type=jnp.float32) + b_ref[...])


def _pre2(ea, w2, b1):
    EB = 8000
    return pl.pallas_call(
        _pre2_body,
        grid=(E // EB,),
        in_specs=[pl.BlockSpec((EB, 16), lambda i: (i, 0)),
                  pl.BlockSpec((16, HID), lambda i: (0, 0)),
                  pl.BlockSpec((1, HID), lambda i: (0, 0))],
        out_specs=pl.BlockSpec((EB, HID), lambda i: (i, 0)),
        out_shape=jax.ShapeDtypeStruct((E, HID), jnp.float32),
    )(ea, w2, b1)


def _combine_body(hacc_ref, deg_ref, w2_ref, b2_ref, h_ref, t1_ref, dist_ref):
    hs = hacc_ref[0] + hacc_ref[1]
    deg = (deg_ref[0, :, :1] + deg_ref[1, :, :1])
    h = jnp.dot(hs, w2_ref[...], preferred_element_type=jnp.float32) \
        + deg * b2_ref[...]
    dist = jnp.where(deg > 0, lax.rsqrt(deg), 0.0)
    h_ref[...] = h
    t1_ref[...] = dist * h
    dist_ref[...] = dist


def _combine(hacc, degp, w2, b2):
    return pl.pallas_call(
        _combine_body,
        grid=(GRID_N,),
        in_specs=[pl.BlockSpec((2, MB, HID), lambda i: (0, i, 0)),
                  pl.BlockSpec((2, MB, HID), lambda i: (0, i, 0)),
                  pl.BlockSpec((HID, HID), lambda i: (0, 0)),
                  pl.BlockSpec((1, HID), lambda i: (0, 0))],
        out_specs=[pl.BlockSpec((MB, HID), lambda i: (i, 0)),
                   pl.BlockSpec((MB, HID), lambda i: (i, 0)),
                   pl.BlockSpec((MB, 1), lambda i: (i, 0))],
        out_shape=[jax.ShapeDtypeStruct((N, HID), jnp.float32),
                   jax.ShapeDtypeStruct((N, HID), jnp.float32),
                   jax.ShapeDtypeStruct((N, 1), jnp.float32)],
    )(hacc, degp, w2, b2)


def _mid_body(u_ref, dist_ref, su_ref, t2_ref):
    u = u_ref[0] + u_ref[1]
    dist = dist_ref[...]
    su = dist * u
    su_ref[...] = su
    t2_ref[...] = dist * su


def _mid(uacc, dist):
    return pl.pallas_call(
        _mid_body,
        grid=(GRID_N,),
        in_specs=[pl.BlockSpec((2, MB, HID), lambda i: (0, i, 0)),
                  pl.BlockSpec((MB, 1), lambda i: (i, 0))],
        out_specs=[pl.BlockSpec((MB, HID), lambda i: (i, 0)),
                   pl.BlockSpec((MB, HID), lambda i: (i, 0))],
        out_shape=[jax.ShapeDtypeStruct((N, HID), jnp.float32),
                   jax.ShapeDtypeStruct((N, HID), jnp.float32)],
    )(uacc, dist)


def _post_relu_body(h_ref, su_ref, v_ref, dist_ref, w_ref, b_ref,
                    hn_ref, tn_ref):
    dist = dist_ref[...]
    sv = dist * (v_ref[0] + v_ref[1])
    out = (jnp.dot(h_ref[...], w_ref[0], preferred_element_type=jnp.float32)
           + jnp.dot(su_ref[...], w_ref[1], preferred_element_type=jnp.float32)
           + jnp.dot(sv, w_ref[2], preferred_element_type=jnp.float32)
           + b_ref[...])
    hn = jnp.maximum(out, 0.0)
    hn_ref[...] = hn
    tn_ref[...] = dist * hn


def _post_final_body(h_ref, su_ref, v_ref, dist_ref, w_ref, b_ref, out_ref):
    dist = dist_ref[...]
    sv = dist * (v_ref[0] + v_ref[1])
    out_ref[...] = (
        jnp.dot(h_ref[...], w_ref[0], preferred_element_type=jnp.float32)
        + jnp.dot(su_ref[...], w_ref[1], preferred_element_type=jnp.float32)
        + jnp.dot(sv, w_ref[2], preferred_element_type=jnp.float32)
        + b_ref[...])


def _post(h, su, vacc, dist, w, b, final):
    in_specs = [pl.BlockSpec((MB, HID), lambda i: (i, 0)),
                pl.BlockSpec((MB, HID), lambda i: (i, 0)),
                pl.BlockSpec((2, MB, HID), lambda i: (0, i, 0)),
                pl.BlockSpec((MB, 1), lambda i: (i, 0)),
                pl.BlockSpec((3, HID, HID), lambda i: (0, 0, 0)),
                pl.BlockSpec((1, HID), lambda i: (0, 0))]
    if final:
        return pl.pallas_call(
            _post_final_body,
            grid=(GRID_N,),
            in_specs=in_specs,
            out_specs=pl.BlockSpec((MB, HID), lambda i: (i, 0)),
            out_shape=jax.ShapeDtypeStruct((N, HID), jnp.float32),
        )(h, su, vacc, dist, w, b)
    return pl.pallas_call(
        _post_relu_body,
        grid=(GRID_N,),
        in_specs=in_specs,
        out_specs=[pl.BlockSpec((MB, HID), lambda i: (i, 0)),
                   pl.BlockSpec((MB, HID), lambda i: (i, 0))],
        out_shape=[jax.ShapeDtypeStruct((N, HID), jnp.float32),
                   jax.ShapeDtypeStruct((N, HID), jnp.float32)],
    )(h, su, vacc, dist, w, b)


def kernel(x, edge_index, edge_attr, ea_w1, ea_b1, ea_w2, ea_b2,
           tag1_w, tag1_b, tag2_w, tag2_b, tag3_w, tag3_b):
    xf = x[:, 4:4 + NFEAT]
    ei = edge_index.astype(jnp.int32)
    row0, col0 = ei[0], ei[1]
    directed = ~jnp.any((row0 == col0[0]) & (col0 == row0[0]))

    pad = EP - 2 * E
    zpad = jnp.zeros((pad,), jnp.int32)
    ar = jnp.arange(E, dtype=jnp.int32)
    gcol = jnp.concatenate([col0, row0, zpad])
    grow = jnp.concatenate([row0, col0, zpad])
    ridx = jnp.concatenate([ar, ar, zpad])
    dummy_rev = DUMMY + (ar % (NACC - N))
    dummy_pad = DUMMY + (jnp.arange(pad, dtype=jnp.int32) % (NACC - N))
    scat = jnp.concatenate([col0, jnp.where(directed, row0, dummy_rev),
                            dummy_pad])
    packed = jnp.stack([gcol, grow, ridx, scat], axis=0)
    ipkm = packed.reshape(4, NCHUNKM, CM).transpose(1, 0, 2)
    ipk2 = packed.reshape(4, NCHUNK, C).transpose(1, 0, 2)

    w01 = jnp.concatenate([ea_w1[:NFEAT], ea_w1[NFEAT:2 * NFEAT]], axis=1)
    PQ = _pre1(xf, w01)
    P = PQ[:, :HID]
    Q = PQ[:, HID:]
    R = _pre2(edge_attr, ea_w1[2 * NFEAT:], ea_b1.reshape(1, HID))

    hacc = _msg_call(ipkm, P, Q, R)
    degp = _degs_call(ipk2)
    h, t, dist = _combine(hacc, degp, ea_w2, ea_b2.reshape(1, HID))

    for (w, b, final) in ((tag1_w, tag1_b, False),
                          (tag2_w, tag2_b, False),
                          (tag3_w, tag3_b, True)):
        uacc = _prop_call(ipk2, t)
        su, t2 = _mid(uacc, dist)
        vacc = _prop_call(ipk2, t2)
        res = _post(h, su, vacc, dist, w, b.reshape(1, HID), final)
        if final:
            return res
        h, t = res


# speedup vs baseline: 2.3708x; 1.1897x over previous
import jax
import jax.numpy as jnp
from jax import lax
from jax.experimental import pallas as pl
from jax.experimental.pallas import tpu as pltpu
from jax.experimental.pallas import tpu_sc as plsc

NFEAT = 128
HID = 128
N = 10000
E = 320000
C = 128
NW = 32
CPT = 157
NCHUNK = NW * CPT
EP = NCHUNK * C
CM = 64
CPTM = EP // (NW * CM)
NCHUNKM = NW * CPTM
NACC = 10112
DUMMY = N
RPT = NACC // 16

MB = 1000
GRID_N = N // MB


def _sc_mesh():
    return plsc.VectorSubcoreMesh(core_axis_name="c", subcore_axis_name="s",
                                  num_cores=2, num_subcores=16)


def _msg_body(ipk, p_hbm, q_hbm, r_hbm, hacc_hbm,
              ibuf, buf, bufp, bufq, hsh, sem):
    cid = lax.axis_index("c")
    sid = lax.axis_index("s")
    wid = cid * 16 + sid
    base = sid * RPT

    zero16 = jnp.zeros((16,), jnp.float32)

    def zrow(i, carry):
        for g in range(HID // 16):
            bufp[i, pl.ds(g * 16, 16)] = zero16
        return carry

    lax.fori_loop(0, CM, zrow, 0)
    for k in range(RPT // CM):
        pltpu.sync_copy(bufp, hsh.at[pl.ds(base + k * CM, CM)])
    rem = RPT % CM
    if rem:
        pltpu.sync_copy(bufp.at[pl.ds(0, rem)],
                        hsh.at[pl.ds(base + (RPT // CM) * CM, rem)])
    plsc.subcore_barrier()

    def chunk(c, carry):
        cidx = wid * CPTM + c
        pltpu.sync_copy(ipk.at[cidx], ibuf)
        rbase = lax.rem(cidx * CM, E)
        d0 = pltpu.async_copy(r_hbm.at[pl.ds(rbase, CM)], buf, sem)
        d1 = pltpu.async_copy(p_hbm.at[ibuf.at[0]], bufp, sem)
        d2 = pltpu.async_copy(q_hbm.at[ibuf.at[1]], bufq, sem)
        d0.wait()
        d1.wait()
        d2.wait()

        def relu_row(i, rc):
            for g in range(HID // 16):
                s = pl.ds(g * 16, 16)
                buf[i, s] = jnp.maximum(buf[i, s] + bufp[i, s] + bufq[i, s],
                                        0.0)
            return rc

        lax.fori_loop(0, CM, relu_row, 0)
        pltpu.sync_copy(buf, hsh.at[ibuf.at[3]], add=True)
        return carry

    lax.fori_loop(0, CPTM, chunk, 0)
    plsc.subcore_barrier()
    pltpu.sync_copy(hsh.at[pl.ds(base, RPT)], hacc_hbm.at[cid, pl.ds(base, RPT)])


def _zero_rows(src, dst, base, n, width_rows):
    for k in range(n // width_rows):
        pltpu.sync_copy(src, dst.at[pl.ds(base + k * width_rows, width_rows)])
    rem = n % width_rows
    if rem:
        pltpu.sync_copy(src.at[pl.ds(0, rem)],
                        dst.at[pl.ds(base + (n // width_rows) * width_rows, rem)])


def _prop_body(ipk4, tab_hbm, acc_hbm, ib0, buf0, zbuf, hsh, gs0):
    cid = lax.axis_index("c")
    sid = lax.axis_index("s")
    wid = cid * 16 + sid
    base = sid * RPT

    zero16 = jnp.zeros((16,), jnp.float32)

    def zrow(i, carry):
        for g in range(HID // 16):
            zbuf[i, pl.ds(g * 16, 16)] = zero16
        return carry

    lax.fori_loop(0, C, zrow, 0)
    _zero_rows(zbuf, hsh, base, RPT, C)
    plsc.subcore_barrier()

    cbase = wid * CPT

    def chunk(c, carry):
        pltpu.sync_copy(ipk4.at[cbase + c], ib0)
        pltpu.async_copy(tab_hbm.at[ib0.at[1]], buf0, gs0).wait()
        pltpu.sync_copy(buf0, hsh.at[ib0.at[3]], add=True)
        return carry

    lax.fori_loop(0, CPT, chunk, 0)
    plsc.subcore_barrier()
    pltpu.sync_copy(hsh.at[pl.ds(base, RPT)], acc_hbm.at[cid, pl.ds(base, RPT)])


def _degs_body(ipk2, deg_hbm, ib0, ib1, ib2, ib3, obuf, dsh,
               is0, is1, is2, is3):
    cid = lax.axis_index("c")
    sid = lax.axis_index("s")
    wid = cid * 16 + sid
    base = sid * RPT
    ibs = (ib0, ib1, ib2, ib3)
    isems = (is0, is1, is2, is3)

    zero16 = jnp.zeros((16,), jnp.float32)
    ones16 = jnp.ones((16,), jnp.float32)

    def zrow(i, carry):
        for g in range(HID // 16):
            obuf[i, pl.ds(g * 16, 16)] = zero16
        return carry

    lax.fori_loop(0, C, zrow, 0)
    _zero_rows(obuf, dsh, base, RPT, C)

    def orow(i, carry):
        for g in range(HID // 16):
            obuf[i, pl.ds(g * 16, 16)] = ones16
        return carry

    lax.fori_loop(0, C, orow, 0)
    plsc.subcore_barrier()

    cbase = wid * CPT

    def chunk(c, carry):
        pltpu.sync_copy(ipk2.at[cbase + c], ibs[0])
        pltpu.sync_copy(obuf, dsh.at[ibs[0].at[3]], add=True)
        return carry

    lax.fori_loop(0, CPT, chunk, 0)
    plsc.subcore_barrier()
    pltpu.sync_copy(dsh.at[pl.ds(base, RPT)], deg_hbm.at[cid, pl.ds(base, RPT)])


def _msg_call(ipk, P, Q, R):
    return pl.kernel(
        _msg_body,
        out_type=jax.ShapeDtypeStruct((2, NACC, HID), jnp.float32),
        mesh=_sc_mesh(),
        scratch_types=[
            pltpu.VMEM((4, CM), jnp.int32),
            pltpu.VMEM((CM, HID), jnp.float32),
            pltpu.VMEM((CM, HID), jnp.float32),
            pltpu.VMEM((CM, HID), jnp.float32),
            pltpu.VMEM_SHARED((NACC, HID), jnp.float32),
            pltpu.SemaphoreType.DMA,
        ],
    )(ipk, P, Q, R)


def _prop_call(ipk2, table):
    return pl.kernel(
        _prop_body,
        out_type=jax.ShapeDtypeStruct((2, NACC, HID), jnp.float32),
        mesh=_sc_mesh(),
        scratch_types=[
            pltpu.VMEM((4, C), jnp.int32),
            pltpu.VMEM((C, HID), jnp.float32),
            pltpu.VMEM((C, HID), jnp.float32),
            pltpu.VMEM_SHARED((NACC, HID), jnp.float32),
            pltpu.SemaphoreType.DMA,
        ],
    )(ipk2, table)


def _degs_call(ipk2):
    return pl.kernel(
        _degs_body,
        out_type=jax.ShapeDtypeStruct((2, NACC, HID), jnp.float32),
        mesh=_sc_mesh(),
        scratch_types=[
            pltpu.VMEM((4, C), jnp.int32),
            pltpu.VMEM((4, C), jnp.int32),
            pltpu.VMEM((4, C), jnp.int32),
            pltpu.VMEM((4, C), jnp.int32),
            pltpu.VMEM((C, HID), jnp.float32),
            pltpu.VMEM_SHARED((NACC, HID), jnp.float32),
            pltpu.SemaphoreType.DMA,
            pltpu.SemaphoreType.DMA,
            pltpu.SemaphoreType.DMA,
            pltpu.SemaphoreType.DMA,
        ],
    )(ipk2)


def _pre1_body(xf_ref, w_ref, o_ref):
    o_ref[...] = jnp.dot(xf_ref[...], w_ref[...],
                         preferred_element_type=jnp.float32)


def _pre1(xf, w01):
    return pl.pallas_call(
        _pre1_body,
        grid=(GRID_N,),
        in_specs=[pl.BlockSpec((MB, NFEAT), lambda i: (i, 0)),
                  pl.BlockSpec((NFEAT, 2 * HID), lambda i: (0, 0))],
        out_specs=pl.BlockSpec((MB, 2 * HID), lambda i: (i, 0)),
        out_shape=jax.ShapeDtypeStruct((N, 2 * HID), jnp.float32),
    )(xf, w01)


def _pre2_body(ea_ref, w_ref, b_ref, r_ref):
    r_ref[...] = (jnp.dot(ea_ref[...], w_ref[...],
                          preferred_element_type=jnp.float32) + b_ref[...])


def _pre2(ea, w2, b1):
    EB = 8000
    return pl.pallas_call(
        _pre2_body,
        grid=(E // EB,),
        in_specs=[pl.BlockSpec((EB, 16), lambda i: (i, 0)),
                  pl.BlockSpec((16, HID), lambda i: (0, 0)),
                  pl.BlockSpec((1, HID), lambda i: (0, 0))],
        out_specs=pl.BlockSpec((EB, HID), lambda i: (i, 0)),
        out_shape=jax.ShapeDtypeStruct((E, HID), jnp.float32),
    )(ea, w2, b1)


def _combine_body(hacc_ref, deg_ref, w2_ref, b2_ref, h_ref, t1_ref, dist_ref):
    hs = hacc_ref[0] + hacc_ref[1]
    deg = (deg_ref[0, :, :1] + deg_ref[1, :, :1])
    h = jnp.dot(hs, w2_ref[...], preferred_element_type=jnp.float32) \
        + deg * b2_ref[...]
    dist = jnp.where(deg > 0, lax.rsqrt(deg), 0.0)
    h_ref[...] = h
    t1_ref[...] = dist * h
    dist_ref[...] = dist


def _combine(hacc, degp, w2, b2):
    return pl.pallas_call(
        _combine_body,
        grid=(GRID_N,),
        in_specs=[pl.BlockSpec((2, MB, HID), lambda i: (0, i, 0)),
                  pl.BlockSpec((2, MB, HID), lambda i: (0, i, 0)),
                  pl.BlockSpec((HID, HID), lambda i: (0, 0)),
                  pl.BlockSpec((1, HID), lambda i: (0, 0))],
        out_specs=[pl.BlockSpec((MB, HID), lambda i: (i, 0)),
                   pl.BlockSpec((MB, HID), lambda i: (i, 0)),
                   pl.BlockSpec((MB, 1), lambda i: (i, 0))],
        out_shape=[jax.ShapeDtypeStruct((N, HID), jnp.float32),
                   jax.ShapeDtypeStruct((N, HID), jnp.float32),
                   jax.ShapeDtypeStruct((N, 1), jnp.float32)],
    )(hacc, degp, w2, b2)


def _mid_body(u_ref, dist_ref, su_ref, t2_ref):
    u = u_ref[0] + u_ref[1]
    dist = dist_ref[...]
    su = dist * u
    su_ref[...] = su
    t2_ref[...] = dist * su


def _mid(uacc, dist):
    return pl.pallas_call(
        _mid_body,
        grid=(GRID_N,),
        in_specs=[pl.BlockSpec((2, MB, HID), lambda i: (0, i, 0)),
                  pl.BlockSpec((MB, 1), lambda i: (i, 0))],
        out_specs=[pl.BlockSpec((MB, HID), lambda i: (i, 0)),
                   pl.BlockSpec((MB, HID), lambda i: (i, 0))],
        out_shape=[jax.ShapeDtypeStruct((N, HID), jnp.float32),
                   jax.ShapeDtypeStruct((N, HID), jnp.float32)],
    )(uacc, dist)


def _post_relu_body(h_ref, su_ref, v_ref, dist_ref, w_ref, b_ref,
                    hn_ref, tn_ref):
    dist = dist_ref[...]
    sv = dist * (v_ref[0] + v_ref[1])
    out = (jnp.dot(h_ref[...], w_ref[0], preferred_element_type=jnp.float32)
           + jnp.dot(su_ref[...], w_ref[1], preferred_element_type=jnp.float32)
           + jnp.dot(sv, w_ref[2], preferred_element_type=jnp.float32)
           + b_ref[...])
    hn = jnp.maximum(out, 0.0)
    hn_ref[...] = hn
    tn_ref[...] = dist * hn


def _post_final_body(h_ref, su_ref, v_ref, dist_ref, w_ref, b_ref, out_ref):
    dist = dist_ref[...]
    sv = dist * (v_ref[0] + v_ref[1])
    out_ref[...] = (
        jnp.dot(h_ref[...], w_ref[0], preferred_element_type=jnp.float32)
        + jnp.dot(su_ref[...], w_ref[1], preferred_element_type=jnp.float32)
        + jnp.dot(sv, w_ref[2], preferred_element_type=jnp.float32)
        + b_ref[...])


def _post(h, su, vacc, dist, w, b, final):
    in_specs = [pl.BlockSpec((MB, HID), lambda i: (i, 0)),
                pl.BlockSpec((MB, HID), lambda i: (i, 0)),
                pl.BlockSpec((2, MB, HID), lambda i: (0, i, 0)),
                pl.BlockSpec((MB, 1), lambda i: (i, 0)),
                pl.BlockSpec((3, HID, HID), lambda i: (0, 0, 0)),
                pl.BlockSpec((1, HID), lambda i: (0, 0))]
    if final:
        return pl.pallas_call(
            _post_final_body,
            grid=(GRID_N,),
            in_specs=in_specs,
            out_specs=pl.BlockSpec((MB, HID), lambda i: (i, 0)),
            out_shape=jax.ShapeDtypeStruct((N, HID), jnp.float32),
        )(h, su, vacc, dist, w, b)
    return pl.pallas_call(
        _post_relu_body,
        grid=(GRID_N,),
        in_specs=in_specs,
        out_specs=[pl.BlockSpec((MB, HID), lambda i: (i, 0)),
                   pl.BlockSpec((MB, HID), lambda i: (i, 0))],
        out_shape=[jax.ShapeDtypeStruct((N, HID), jnp.float32),
                   jax.ShapeDtypeStruct((N, HID), jnp.float32)],
    )(h, su, vacc, dist, w, b)


def kernel(x, edge_index, edge_attr, ea_w1, ea_b1, ea_w2, ea_b2,
           tag1_w, tag1_b, tag2_w, tag2_b, tag3_w, tag3_b):
    xf = x[:, 4:4 + NFEAT]
    ei = edge_index.astype(jnp.int32)
    row0, col0 = ei[0], ei[1]
    directed = ~jnp.any((row0 == col0[0]) & (col0 == row0[0]))

    pad = EP - 2 * E
    arp = jnp.arange(pad, dtype=jnp.int32)
    zpad = arp % N
    ar = jnp.arange(E, dtype=jnp.int32)
    gcol = jnp.concatenate([col0, row0, zpad])
    grow = jnp.concatenate([row0, col0, zpad])
    ridx = jnp.concatenate([ar, ar, arp % E])
    dummy_rev = DUMMY + (ar % (NACC - N))
    dummy_pad = DUMMY + (arp % (NACC - N))
    scat = jnp.concatenate([col0, jnp.where(directed, row0, dummy_rev),
                            dummy_pad])
    packed = jnp.stack([gcol, grow, ridx, scat], axis=0)
    ipkm = packed.reshape(4, NCHUNKM, CM).transpose(1, 0, 2)
    ipk2 = packed.reshape(4, NCHUNK, C).transpose(1, 0, 2)

    w01 = jnp.concatenate([ea_w1[:NFEAT], ea_w1[NFEAT:2 * NFEAT]], axis=1)
    PQ = _pre1(xf, w01)
    P = PQ[:, :HID]
    Q = PQ[:, HID:]
    R = _pre2(edge_attr, ea_w1[2 * NFEAT:], ea_b1.reshape(1, HID))

    hacc = _msg_call(ipkm, P, Q, R)
    degp = _degs_call(ipk2)
    h, t, dist = _combine(hacc, degp, ea_w2, ea_b2.reshape(1, HID))

    for (w, b, final) in ((tag1_w, tag1_b, False),
                          (tag2_w, tag2_b, False),
                          (tag3_w, tag3_b, True)):
        uacc = _prop_call(ipk2, t)
        su, t2 = _mid(uacc, dist)
        vacc = _prop_call(ipk2, t2)
        res = _post(h, su, vacc, dist, w, b.reshape(1, HID), final)
        if final:
            return res
        h, t = res


# speedup vs baseline: 2.5244x; 1.0648x over previous
import jax
import jax.numpy as jnp
from jax import lax
from jax.experimental import pallas as pl
from jax.experimental.pallas import tpu as pltpu
from jax.experimental.pallas import tpu_sc as plsc

NFEAT = 128
HID = 128
N = 10000
E = 320000
C = 128
NW = 32
CPT = 157
NCHUNK = NW * CPT
EP = NCHUNK * C
CM = 32
CPTM = EP // (NW * CM)
NCHUNKM = NW * CPTM
NACC = 10112
DUMMY = N
RPT = NACC // 16

MB = 1000
GRID_N = N // MB


def _sc_mesh():
    return plsc.VectorSubcoreMesh(core_axis_name="c", subcore_axis_name="s",
                                  num_cores=2, num_subcores=16)


def _msg_body(ipk, p_hbm, q_hbm, r_hbm, hacc_hbm,
              ib0, ib1, br0, bp0, bq0, br1, bp1, bq1, hsh, gs0, gs1):
    cid = lax.axis_index("c")
    sid = lax.axis_index("s")
    wid = cid * 16 + sid
    base = sid * RPT
    ibs = (ib0, ib1)
    brs = (br0, br1)
    bps = (bp0, bp1)
    bqs = (bq0, bq1)
    gsems = (gs0, gs1)

    zero16 = jnp.zeros((16,), jnp.float32)

    def zrow(i, carry):
        for g in range(HID // 16):
            bp0[i, pl.ds(g * 16, 16)] = zero16
        return carry

    lax.fori_loop(0, CM, zrow, 0)
    for k in range(RPT // CM):
        pltpu.sync_copy(bp0, hsh.at[pl.ds(base + k * CM, CM)])
    rem = RPT % CM
    if rem:
        pltpu.sync_copy(bp0.at[pl.ds(0, rem)],
                        hsh.at[pl.ds(base + (RPT // CM) * CM, rem)])
    plsc.subcore_barrier()

    cbase = wid * CPTM

    def fetch(c, s):
        pltpu.sync_copy(ipk.at[cbase + c], ibs[s])
        rbase = lax.rem((cbase + c) * CM, E)
        pltpu.async_copy(r_hbm.at[pl.ds(rbase, CM)], brs[s], gsems[s])
        pltpu.async_copy(p_hbm.at[ibs[s].at[0]], bps[s], gsems[s])
        pltpu.async_copy(q_hbm.at[ibs[s].at[1]], bqs[s], gsems[s])

    def drain_relu_scatter(c, s):
        rbase = lax.rem((cbase + c) * CM, E)
        pltpu.make_async_copy(r_hbm.at[pl.ds(rbase, CM)], brs[s],
                              gsems[s]).wait()
        pltpu.make_async_copy(p_hbm.at[ibs[s].at[0]], bps[s], gsems[s]).wait()
        pltpu.make_async_copy(q_hbm.at[ibs[s].at[1]], bqs[s], gsems[s]).wait()
        br, bp, bq = brs[s], bps[s], bqs[s]

        def relu_row(i, rc):
            for g in range(HID // 16):
                sl = pl.ds(g * 16, 16)
                br[i, sl] = jnp.maximum(br[i, sl] + bp[i, sl] + bq[i, sl],
                                        0.0)
            return rc

        lax.fori_loop(0, CM, relu_row, 0)
        pltpu.sync_copy(br, hsh.at[ibs[s].at[3]], add=True)

    fetch(0, 0)

    def pair(t, carry):
        c = t * 2
        fetch(c + 1, 1)
        drain_relu_scatter(c, 0)

        @pl.when(c + 2 < CPTM)
        def _():
            fetch(c + 2, 0)

        drain_relu_scatter(c + 1, 1)
        return carry

    lax.fori_loop(0, CPTM // 2, pair, 0)
    plsc.subcore_barrier()
    pltpu.sync_copy(hsh.at[pl.ds(base, RPT)], hacc_hbm.at[cid, pl.ds(base, RPT)])


def _zero_rows(src, dst, base, n, width_rows):
    for k in range(n // width_rows):
        pltpu.sync_copy(src, dst.at[pl.ds(base + k * width_rows, width_rows)])
    rem = n % width_rows
    if rem:
        pltpu.sync_copy(src.at[pl.ds(0, rem)],
                        dst.at[pl.ds(base + (n // width_rows) * width_rows, rem)])


def _prop_body(ipk4, tab_hbm, acc_hbm, ib0, buf0, zbuf, hsh, gs0):
    cid = lax.axis_index("c")
    sid = lax.axis_index("s")
    wid = cid * 16 + sid
    base = sid * RPT

    zero16 = jnp.zeros((16,), jnp.float32)

    def zrow(i, carry):
        for g in range(HID // 16):
            zbuf[i, pl.ds(g * 16, 16)] = zero16
        return carry

    lax.fori_loop(0, C, zrow, 0)
    _zero_rows(zbuf, hsh, base, RPT, C)
    plsc.subcore_barrier()

    cbase = wid * CPT

    def chunk(c, carry):
        pltpu.sync_copy(ipk4.at[cbase + c], ib0)
        pltpu.async_copy(tab_hbm.at[ib0.at[1]], buf0, gs0).wait()
        pltpu.sync_copy(buf0, hsh.at[ib0.at[3]], add=True)
        return carry

    lax.fori_loop(0, CPT, chunk, 0)
    plsc.subcore_barrier()
    pltpu.sync_copy(hsh.at[pl.ds(base, RPT)], acc_hbm.at[cid, pl.ds(base, RPT)])


def _degs_body(ipk2, deg_hbm, ib0, ib1, ib2, ib3, obuf, dsh,
               is0, is1, is2, is3):
    cid = lax.axis_index("c")
    sid = lax.axis_index("s")
    wid = cid * 16 + sid
    base = sid * RPT
    ibs = (ib0, ib1, ib2, ib3)
    isems = (is0, is1, is2, is3)

    zero16 = jnp.zeros((16,), jnp.float32)
    ones16 = jnp.ones((16,), jnp.float32)

    def zrow(i, carry):
        for g in range(HID // 16):
            obuf[i, pl.ds(g * 16, 16)] = zero16
        return carry

    lax.fori_loop(0, C, zrow, 0)
    _zero_rows(obuf, dsh, base, RPT, C)

    def orow(i, carry):
        for g in range(HID // 16):
            obuf[i, pl.ds(g * 16, 16)] = ones16
        return carry

    lax.fori_loop(0, C, orow, 0)
    plsc.subcore_barrier()

    cbase = wid * CPT

    def chunk(c, carry):
        pltpu.sync_copy(ipk2.at[cbase + c], ibs[0])
        pltpu.sync_copy(obuf, dsh.at[ibs[0].at[3]], add=True)
        return carry

    lax.fori_loop(0, CPT, chunk, 0)
    plsc.subcore_barrier()
    pltpu.sync_copy(dsh.at[pl.ds(base, RPT)], deg_hbm.at[cid, pl.ds(base, RPT)])


def _msg_call(ipk, P, Q, R):
    return pl.kernel(
        _msg_body,
        out_type=jax.ShapeDtypeStruct((2, NACC, HID), jnp.float32),
        mesh=_sc_mesh(),
        scratch_types=[
            pltpu.VMEM((4, CM), jnp.int32),
            pltpu.VMEM((4, CM), jnp.int32),
            pltpu.VMEM((CM, HID), jnp.float32),
            pltpu.VMEM((CM, HID), jnp.float32),
            pltpu.VMEM((CM, HID), jnp.float32),
            pltpu.VMEM((CM, HID), jnp.float32),
            pltpu.VMEM((CM, HID), jnp.float32),
            pltpu.VMEM((CM, HID), jnp.float32),
            pltpu.VMEM_SHARED((NACC, HID), jnp.float32),
            pltpu.SemaphoreType.DMA,
            pltpu.SemaphoreType.DMA,
        ],
    )(ipk, P, Q, R)


def _prop_call(ipk2, table):
    return pl.kernel(
        _prop_body,
        out_type=jax.ShapeDtypeStruct((2, NACC, HID), jnp.float32),
        mesh=_sc_mesh(),
        scratch_types=[
            pltpu.VMEM((4, C), jnp.int32),
            pltpu.VMEM((C, HID), jnp.float32),
            pltpu.VMEM((C, HID), jnp.float32),
            pltpu.VMEM_SHARED((NACC, HID), jnp.float32),
            pltpu.SemaphoreType.DMA,
        ],
    )(ipk2, table)


def _degs_call(ipk2):
    return pl.kernel(
        _degs_body,
        out_type=jax.ShapeDtypeStruct((2, NACC, HID), jnp.float32),
        mesh=_sc_mesh(),
        scratch_types=[
            pltpu.VMEM((4, C), jnp.int32),
            pltpu.VMEM((4, C), jnp.int32),
            pltpu.VMEM((4, C), jnp.int32),
            pltpu.VMEM((4, C), jnp.int32),
            pltpu.VMEM((C, HID), jnp.float32),
            pltpu.VMEM_SHARED((NACC, HID), jnp.float32),
            pltpu.SemaphoreType.DMA,
            pltpu.SemaphoreType.DMA,
            pltpu.SemaphoreType.DMA,
            pltpu.SemaphoreType.DMA,
        ],
    )(ipk2)


def _pre1_body(xf_ref, w_ref, o_ref):
    o_ref[...] = jnp.dot(xf_ref[...], w_ref[...],
                         preferred_element_type=jnp.float32)


def _pre1(xf, w01):
    return pl.pallas_call(
        _pre1_body,
        grid=(GRID_N,),
        in_specs=[pl.BlockSpec((MB, NFEAT), lambda i: (i, 0)),
                  pl.BlockSpec((NFEAT, 2 * HID), lambda i: (0, 0))],
        out_specs=pl.BlockSpec((MB, 2 * HID), lambda i: (i, 0)),
        out_shape=jax.ShapeDtypeStruct((N, 2 * HID), jnp.float32),
    )(xf, w01)


def _pre2_body(ea_ref, w_ref, b_ref, r_ref):
    r_ref[...] = (jnp.dot(ea_ref[...], w_ref[...],
                          preferred_element_type=jnp.float32) + b_ref[...])


def _pre2(ea, w2, b1):
    EB = 8000
    return pl.pallas_call(
        _pre2_body,
        grid=(E // EB,),
        in_specs=[pl.BlockSpec((EB, 16), lambda i: (i, 0)),
                  pl.BlockSpec((16, HID), lambda i: (0, 0)),
                  pl.BlockSpec((1, HID), lambda i: (0, 0))],
        out_specs=pl.BlockSpec((EB, HID), lambda i: (i, 0)),
        out_shape=jax.ShapeDtypeStruct((E, HID), jnp.float32),
    )(ea, w2, b1)


def _combine_body(hacc_ref, deg_ref, w2_ref, b2_ref, h_ref, t1_ref, dist_ref):
    hs = hacc_ref[0] + hacc_ref[1]
    deg = (deg_ref[0, :, :1] + deg_ref[1, :, :1])
    h = jnp.dot(hs, w2_ref[...], preferred_element_type=jnp.float32) \
        + deg * b2_ref[...]
    dist = jnp.where(deg > 0, lax.rsqrt(deg), 0.0)
    h_ref[...] = h
    t1_ref[...] = dist * h
    dist_ref[...] = dist


def _combine(hacc, degp, w2, b2):
    return pl.pallas_call(
        _combine_body,
        grid=(GRID_N,),
        in_specs=[pl.BlockSpec((2, MB, HID), lambda i: (0, i, 0)),
                  pl.BlockSpec((2, MB, HID), lambda i: (0, i, 0)),
                  pl.BlockSpec((HID, HID), lambda i: (0, 0)),
                  pl.BlockSpec((1, HID), lambda i: (0, 0))],
        out_specs=[pl.BlockSpec((MB, HID), lambda i: (i, 0)),
                   pl.BlockSpec((MB, HID), lambda i: (i, 0)),
                   pl.BlockSpec((MB, 1), lambda i: (i, 0))],
        out_shape=[jax.ShapeDtypeStruct((N, HID), jnp.float32),
                   jax.ShapeDtypeStruct((N, HID), jnp.float32),
                   jax.ShapeDtypeStruct((N, 1), jnp.float32)],
    )(hacc, degp, w2, b2)


def _mid_body(u_ref, dist_ref, su_ref, t2_ref):
    u = u_ref[0] + u_ref[1]
    dist = dist_ref[...]
    su = dist * u
    su_ref[...] = su
    t2_ref[...] = dist * su


def _mid(uacc, dist):
    return pl.pallas_call(
        _mid_body,
        grid=(GRID_N,),
        in_specs=[pl.BlockSpec((2, MB, HID), lambda i: (0, i, 0)),
                  pl.BlockSpec((MB, 1), lambda i: (i, 0))],
        out_specs=[pl.BlockSpec((MB, HID), lambda i: (i, 0)),
                   pl.BlockSpec((MB, HID), lambda i: (i, 0))],
        out_shape=[jax.ShapeDtypeStruct((N, HID), jnp.float32),
                   jax.ShapeDtypeStruct((N, HID), jnp.float32)],
    )(uacc, dist)


def _post_relu_body(h_ref, su_ref, v_ref, dist_ref, w_ref, b_ref,
                    hn_ref, tn_ref):
    dist = dist_ref[...]
    sv = dist * (v_ref[0] + v_ref[1])
    out = (jnp.dot(h_ref[...], w_ref[0], preferred_element_type=jnp.float32)
           + jnp.dot(su_ref[...], w_ref[1], preferred_element_type=jnp.float32)
           + jnp.dot(sv, w_ref[2], preferred_element_type=jnp.float32)
           + b_ref[...])
    hn = jnp.maximum(out, 0.0)
    hn_ref[...] = hn
    tn_ref[...] = dist * hn


def _post_final_body(h_ref, su_ref, v_ref, dist_ref, w_ref, b_ref, out_ref):
    dist = dist_ref[...]
    sv = dist * (v_ref[0] + v_ref[1])
    out_ref[...] = (
        jnp.dot(h_ref[...], w_ref[0], preferred_element_type=jnp.float32)
        + jnp.dot(su_ref[...], w_ref[1], preferred_element_type=jnp.float32)
        + jnp.dot(sv, w_ref[2], preferred_element_type=jnp.float32)
        + b_ref[...])


def _post(h, su, vacc, dist, w, b, final):
    in_specs = [pl.BlockSpec((MB, HID), lambda i: (i, 0)),
                pl.BlockSpec((MB, HID), lambda i: (i, 0)),
                pl.BlockSpec((2, MB, HID), lambda i: (0, i, 0)),
                pl.BlockSpec((MB, 1), lambda i: (i, 0)),
                pl.BlockSpec((3, HID, HID), lambda i: (0, 0, 0)),
                pl.BlockSpec((1, HID), lambda i: (0, 0))]
    if final:
        return pl.pallas_call(
            _post_final_body,
            grid=(GRID_N,),
            in_specs=in_specs,
            out_specs=pl.BlockSpec((MB, HID), lambda i: (i, 0)),
            out_shape=jax.ShapeDtypeStruct((N, HID), jnp.float32),
        )(h, su, vacc, dist, w, b)
    return pl.pallas_call(
        _post_relu_body,
        grid=(GRID_N,),
        in_specs=in_specs,
        out_specs=[pl.BlockSpec((MB, HID), lambda i: (i, 0)),
                   pl.BlockSpec((MB, HID), lambda i: (i, 0))],
        out_shape=[jax.ShapeDtypeStruct((N, HID), jnp.float32),
                   jax.ShapeDtypeStruct((N, HID), jnp.float32)],
    )(h, su, vacc, dist, w, b)


def kernel(x, edge_index, edge_attr, ea_w1, ea_b1, ea_w2, ea_b2,
           tag1_w, tag1_b, tag2_w, tag2_b, tag3_w, tag3_b):
    xf = x[:, 4:4 + NFEAT]
    ei = edge_index.astype(jnp.int32)
    row0, col0 = ei[0], ei[1]
    directed = ~jnp.any((row0 == col0[0]) & (col0 == row0[0]))

    pad = EP - 2 * E
    arp = jnp.arange(pad, dtype=jnp.int32)
    zpad = arp % N
    ar = jnp.arange(E, dtype=jnp.int32)
    gcol = jnp.concatenate([col0, row0, zpad])
    grow = jnp.concatenate([row0, col0, zpad])
    ridx = jnp.concatenate([ar, ar, arp % E])
    dummy_rev = DUMMY + (ar % (NACC - N))
    dummy_pad = DUMMY + (arp % (NACC - N))
    scat = jnp.concatenate([col0, jnp.where(directed, row0, dummy_rev),
                            dummy_pad])
    packed = jnp.stack([gcol, grow, ridx, scat], axis=0)
    ipkm = packed.reshape(4, NCHUNKM, CM).transpose(1, 0, 2)
    ipk2 = packed.reshape(4, NCHUNK, C).transpose(1, 0, 2)

    w01 = jnp.concatenate([ea_w1[:NFEAT], ea_w1[NFEAT:2 * NFEAT]], axis=1)
    PQ = _pre1(xf, w01)
    P = PQ[:, :HID]
    Q = PQ[:, HID:]
    R = _pre2(edge_attr, ea_w1[2 * NFEAT:], ea_b1.reshape(1, HID))

    hacc = _msg_call(ipkm, P, Q, R)
    degp = _degs_call(ipk2)
    h, t, dist = _combine(hacc, degp, ea_w2, ea_b2.reshape(1, HID))

    for (w, b, final) in ((tag1_w, tag1_b, False),
                          (tag2_w, tag2_b, False),
                          (tag3_w, tag3_b, True)):
        uacc = _prop_call(ipk2, t)
        su, t2 = _mid(uacc, dist)
        vacc = _prop_call(ipk2, t2)
        res = _post(h, su, vacc, dist, w, b.reshape(1, HID), final)
        if final:
            return res
        h, t = res


# speedup vs baseline: 3.4981x; 1.3857x over previous
import jax
import jax.numpy as jnp
from jax import lax
from jax.experimental import pallas as pl
from jax.experimental.pallas import tpu as pltpu
from jax.experimental.pallas import tpu_sc as plsc

NFEAT = 128
HID = 128
N = 10000
E = 320000
C = 128
NW = 32
CPT = 157
NCHUNK = NW * CPT
EP = NCHUNK * C
CM = 32
CPTM = EP // (NW * CM)
NCHUNKM = NW * CPTM
NACC = 10112
DUMMY = N
RPT = NACC // 16

MB = 1000
GRID_N = N // MB


def _sc_mesh():
    return plsc.VectorSubcoreMesh(core_axis_name="c", subcore_axis_name="s",
                                  num_cores=2, num_subcores=16)


def _msg_body(ipk, p_hbm, q_hbm, r_hbm, hacc_hbm,
              ib0, ib1, br0, bp0, bq0, br1, bp1, bq1, hsh, gs0, gs1):
    cid = lax.axis_index("c")
    sid = lax.axis_index("s")
    wid = cid * 16 + sid
    base = sid * RPT
    ibs = (ib0, ib1)
    brs = (br0, br1)
    bps = (bp0, bp1)
    bqs = (bq0, bq1)
    gsems = (gs0, gs1)

    zero16 = jnp.zeros((16,), jnp.float32)

    def zrow(i, carry):
        for g in range(HID // 16):
            bp0[i, pl.ds(g * 16, 16)] = zero16
        return carry

    lax.fori_loop(0, CM, zrow, 0)
    for k in range(RPT // CM):
        pltpu.sync_copy(bp0, hsh.at[pl.ds(base + k * CM, CM)])
    rem = RPT % CM
    if rem:
        pltpu.sync_copy(bp0.at[pl.ds(0, rem)],
                        hsh.at[pl.ds(base + (RPT // CM) * CM, rem)])
    plsc.subcore_barrier()

    cbase = wid * CPTM

    def fetch(c, s):
        pltpu.sync_copy(ipk.at[cbase + c], ibs[s])
        rbase = lax.rem((cbase + c) * CM, E)
        pltpu.async_copy(r_hbm.at[pl.ds(rbase, CM)], brs[s], gsems[s])
        pltpu.async_copy(p_hbm.at[ibs[s].at[0]], bps[s], gsems[s])
        pltpu.async_copy(q_hbm.at[ibs[s].at[1]], bqs[s], gsems[s])

    def drain_relu_scatter(c, s):
        rbase = lax.rem((cbase + c) * CM, E)
        pltpu.make_async_copy(r_hbm.at[pl.ds(rbase, CM)], brs[s],
                              gsems[s]).wait()
        pltpu.make_async_copy(p_hbm.at[ibs[s].at[0]], bps[s], gsems[s]).wait()
        pltpu.make_async_copy(q_hbm.at[ibs[s].at[1]], bqs[s], gsems[s]).wait()
        br, bp, bq = brs[s], bps[s], bqs[s]

        def relu_row(i, rc):
            for g in range(HID // 16):
                sl = pl.ds(g * 16, 16)
                br[i, sl] = jnp.maximum(br[i, sl] + bp[i, sl] + bq[i, sl],
                                        0.0)
            return rc

        lax.fori_loop(0, CM, relu_row, 0)
        pltpu.sync_copy(br, hsh.at[ibs[s].at[3]], add=True)

    fetch(0, 0)

    def pair(t, carry):
        c = t * 2
        fetch(c + 1, 1)
        drain_relu_scatter(c, 0)

        @pl.when(c + 2 < CPTM)
        def _():
            fetch(c + 2, 0)

        drain_relu_scatter(c + 1, 1)
        return carry

    lax.fori_loop(0, CPTM // 2, pair, 0)
    plsc.subcore_barrier()
    pltpu.sync_copy(hsh.at[pl.ds(base, RPT)], hacc_hbm.at[cid, pl.ds(base, RPT)])


def _zero_rows(src, dst, base, n, width_rows):
    for k in range(n // width_rows):
        pltpu.sync_copy(src, dst.at[pl.ds(base + k * width_rows, width_rows)])
    rem = n % width_rows
    if rem:
        pltpu.sync_copy(src.at[pl.ds(0, rem)],
                        dst.at[pl.ds(base + (n // width_rows) * width_rows, rem)])


def _prop_body(ipk4, tab_hbm, acc_hbm, ib0, ib1, buf0, buf1, hsh, gs0, gs1):
    cid = lax.axis_index("c")
    sid = lax.axis_index("s")
    wid = cid * 16 + sid
    base = sid * RPT
    ibs = (ib0, ib1)
    bufs = (buf0, buf1)
    gsems = (gs0, gs1)

    zero16 = jnp.zeros((16,), jnp.float32)

    def zrow(i, carry):
        for g in range(HID // 16):
            buf0[i, pl.ds(g * 16, 16)] = zero16
        return carry

    lax.fori_loop(0, C, zrow, 0)
    _zero_rows(buf0, hsh, base, RPT, C)
    plsc.subcore_barrier()

    cbase = wid * CPT

    def fetch(c, s):
        pltpu.sync_copy(ipk4.at[cbase + c], ibs[s])
        pltpu.async_copy(tab_hbm.at[ibs[s].at[1]], bufs[s], gsems[s])

    def drain_scatter(s):
        pltpu.make_async_copy(tab_hbm.at[ibs[s].at[1]], bufs[s],
                              gsems[s]).wait()
        pltpu.sync_copy(bufs[s], hsh.at[ibs[s].at[3]], add=True)

    fetch(0, 0)

    def pair(t, carry):
        c = t * 2

        @pl.when(c + 1 < CPT)
        def _():
            fetch(c + 1, 1)

        drain_scatter(0)

        @pl.when(c + 2 < CPT)
        def _():
            fetch(c + 2, 0)

        @pl.when(c + 1 < CPT)
        def _():
            drain_scatter(1)

        return carry

    lax.fori_loop(0, (CPT + 1) // 2, pair, 0)
    plsc.subcore_barrier()
    pltpu.sync_copy(hsh.at[pl.ds(base, RPT)], acc_hbm.at[cid, pl.ds(base, RPT)])


def _degs_body(ipk2, deg_hbm, ib0, ib1, ib2, ib3, obuf, dsh,
               is0, is1, is2, is3):
    cid = lax.axis_index("c")
    sid = lax.axis_index("s")
    wid = cid * 16 + sid
    base = sid * RPT
    ibs = (ib0, ib1, ib2, ib3)
    isems = (is0, is1, is2, is3)

    zero16 = jnp.zeros((16,), jnp.float32)
    ones16 = jnp.ones((16,), jnp.float32)

    def zrow(i, carry):
        for g in range(HID // 16):
            obuf[i, pl.ds(g * 16, 16)] = zero16
        return carry

    lax.fori_loop(0, C, zrow, 0)
    _zero_rows(obuf, dsh, base, RPT, C)

    def orow(i, carry):
        for g in range(HID // 16):
            obuf[i, pl.ds(g * 16, 16)] = ones16
        return carry

    lax.fori_loop(0, C, orow, 0)
    plsc.subcore_barrier()

    cbase = wid * CPT

    def chunk(c, carry):
        pltpu.sync_copy(ipk2.at[cbase + c], ibs[0])
        pltpu.sync_copy(obuf, dsh.at[ibs[0].at[3]], add=True)
        return carry

    lax.fori_loop(0, CPT, chunk, 0)
    plsc.subcore_barrier()
    pltpu.sync_copy(dsh.at[pl.ds(base, RPT)], deg_hbm.at[cid, pl.ds(base, RPT)])


def _msg_call(ipk, P, Q, R):
    return pl.kernel(
        _msg_body,
        out_type=jax.ShapeDtypeStruct((2, NACC, HID), jnp.float32),
        mesh=_sc_mesh(),
        scratch_types=[
            pltpu.VMEM((4, CM), jnp.int32),
            pltpu.VMEM((4, CM), jnp.int32),
            pltpu.VMEM((CM, HID), jnp.float32),
            pltpu.VMEM((CM, HID), jnp.float32),
            pltpu.VMEM((CM, HID), jnp.float32),
            pltpu.VMEM((CM, HID), jnp.float32),
            pltpu.VMEM((CM, HID), jnp.float32),
            pltpu.VMEM((CM, HID), jnp.float32),
            pltpu.VMEM_SHARED((NACC, HID), jnp.float32),
            pltpu.SemaphoreType.DMA,
            pltpu.SemaphoreType.DMA,
        ],
    )(ipk, P, Q, R)


def _prop_call(ipk2, table):
    return pl.kernel(
        _prop_body,
        out_type=jax.ShapeDtypeStruct((2, NACC, HID), jnp.float32),
        mesh=_sc_mesh(),
        scratch_types=[
            pltpu.VMEM((4, C), jnp.int32),
            pltpu.VMEM((4, C), jnp.int32),
            pltpu.VMEM((C, HID), jnp.float32),
            pltpu.VMEM((C, HID), jnp.float32),
            pltpu.VMEM_SHARED((NACC, HID), jnp.float32),
            pltpu.SemaphoreType.DMA,
            pltpu.SemaphoreType.DMA,
        ],
    )(ipk2, table)


def _degs_call(ipk2):
    return pl.kernel(
        _degs_body,
        out_type=jax.ShapeDtypeStruct((2, NACC, HID), jnp.float32),
        mesh=_sc_mesh(),
        scratch_types=[
            pltpu.VMEM((4, C), jnp.int32),
            pltpu.VMEM((4, C), jnp.int32),
            pltpu.VMEM((4, C), jnp.int32),
            pltpu.VMEM((4, C), jnp.int32),
            pltpu.VMEM((C, HID), jnp.float32),
            pltpu.VMEM_SHARED((NACC, HID), jnp.float32),
            pltpu.SemaphoreType.DMA,
            pltpu.SemaphoreType.DMA,
            pltpu.SemaphoreType.DMA,
            pltpu.SemaphoreType.DMA,
        ],
    )(ipk2)


def _pre1_body(xf_ref, w_ref, o_ref):
    o_ref[...] = jnp.dot(xf_ref[...], w_ref[...],
                         preferred_element_type=jnp.float32)


def _pre1(xf, w01):
    return pl.pallas_call(
        _pre1_body,
        grid=(GRID_N,),
        in_specs=[pl.BlockSpec((MB, NFEAT), lambda i: (i, 0)),
                  pl.BlockSpec((NFEAT, 2 * HID), lambda i: (0, 0))],
        out_specs=pl.BlockSpec((MB, 2 * HID), lambda i: (i, 0)),
        out_shape=jax.ShapeDtypeStruct((N, 2 * HID), jnp.float32),
    )(xf, w01)


def _pre2_body(ea_ref, w_ref, b_ref, r_ref):
    r_ref[...] = (jnp.dot(ea_ref[...], w_ref[...],
                          preferred_element_type=jnp.float32) + b_ref[...])


def _pre2(ea, w2, b1):
    EB = 8000
    return pl.pallas_call(
        _pre2_body,
        grid=(E // EB,),
        in_specs=[pl.BlockSpec((EB, 16), lambda i: (i, 0)),
                  pl.BlockSpec((16, HID), lambda i: (0, 0)),
                  pl.BlockSpec((1, HID), lambda i: (0, 0))],
        out_specs=pl.BlockSpec((EB, HID), lambda i: (i, 0)),
        out_shape=jax.ShapeDtypeStruct((E, HID), jnp.float32),
    )(ea, w2, b1)


def _combine_body(hacc_ref, deg_ref, w2_ref, b2_ref, h_ref, t1_ref, dist_ref):
    hs = hacc_ref[0] + hacc_ref[1]
    deg = (deg_ref[0, :, :1] + deg_ref[1, :, :1])
    h = jnp.dot(hs, w2_ref[...], preferred_element_type=jnp.float32) \
        + deg * b2_ref[...]
    dist = jnp.where(deg > 0, lax.rsqrt(deg), 0.0)
    h_ref[...] = h
    t1_ref[...] = dist * h
    dist_ref[...] = dist


def _combine(hacc, degp, w2, b2):
    return pl.pallas_call(
        _combine_body,
        grid=(GRID_N,),
        in_specs=[pl.BlockSpec((2, MB, HID), lambda i: (0, i, 0)),
                  pl.BlockSpec((2, MB, HID), lambda i: (0, i, 0)),
                  pl.BlockSpec((HID, HID), lambda i: (0, 0)),
                  pl.BlockSpec((1, HID), lambda i: (0, 0))],
        out_specs=[pl.BlockSpec((MB, HID), lambda i: (i, 0)),
                   pl.BlockSpec((MB, HID), lambda i: (i, 0)),
                   pl.BlockSpec((MB, 1), lambda i: (i, 0))],
        out_shape=[jax.ShapeDtypeStruct((N, HID), jnp.float32),
                   jax.ShapeDtypeStruct((N, HID), jnp.float32),
                   jax.ShapeDtypeStruct((N, 1), jnp.float32)],
    )(hacc, degp, w2, b2)


def _mid_body(u_ref, dist_ref, su_ref, t2_ref):
    u = u_ref[0] + u_ref[1]
    dist = dist_ref[...]
    su = dist * u
    su_ref[...] = su
    t2_ref[...] = dist * su


def _mid(uacc, dist):
    return pl.pallas_call(
        _mid_body,
        grid=(GRID_N,),
        in_specs=[pl.BlockSpec((2, MB, HID), lambda i: (0, i, 0)),
                  pl.BlockSpec((MB, 1), lambda i: (i, 0))],
        out_specs=[pl.BlockSpec((MB, HID), lambda i: (i, 0)),
                   pl.BlockSpec((MB, HID), lambda i: (i, 0))],
        out_shape=[jax.ShapeDtypeStruct((N, HID), jnp.float32),
                   jax.ShapeDtypeStruct((N, HID), jnp.float32)],
    )(uacc, dist)


def _post_relu_body(h_ref, su_ref, v_ref, dist_ref, w_ref, b_ref,
                    hn_ref, tn_ref):
    dist = dist_ref[...]
    sv = dist * (v_ref[0] + v_ref[1])
    out = (jnp.dot(h_ref[...], w_ref[0], preferred_element_type=jnp.float32)
           + jnp.dot(su_ref[...], w_ref[1], preferred_element_type=jnp.float32)
           + jnp.dot(sv, w_ref[2], preferred_element_type=jnp.float32)
           + b_ref[...])
    hn = jnp.maximum(out, 0.0)
    hn_ref[...] = hn
    tn_ref[...] = dist * hn


def _post_final_body(h_ref, su_ref, v_ref, dist_ref, w_ref, b_ref, out_ref):
    dist = dist_ref[...]
    sv = dist * (v_ref[0] + v_ref[1])
    out_ref[...] = (
        jnp.dot(h_ref[...], w_ref[0], preferred_element_type=jnp.float32)
        + jnp.dot(su_ref[...], w_ref[1], preferred_element_type=jnp.float32)
        + jnp.dot(sv, w_ref[2], preferred_element_type=jnp.float32)
        + b_ref[...])


def _post(h, su, vacc, dist, w, b, final):
    in_specs = [pl.BlockSpec((MB, HID), lambda i: (i, 0)),
                pl.BlockSpec((MB, HID), lambda i: (i, 0)),
                pl.BlockSpec((2, MB, HID), lambda i: (0, i, 0)),
                pl.BlockSpec((MB, 1), lambda i: (i, 0)),
                pl.BlockSpec((3, HID, HID), lambda i: (0, 0, 0)),
                pl.BlockSpec((1, HID), lambda i: (0, 0))]
    if final:
        return pl.pallas_call(
            _post_final_body,
            grid=(GRID_N,),
            in_specs=in_specs,
            out_specs=pl.BlockSpec((MB, HID), lambda i: (i, 0)),
            out_shape=jax.ShapeDtypeStruct((N, HID), jnp.float32),
        )(h, su, vacc, dist, w, b)
    return pl.pallas_call(
        _post_relu_body,
        grid=(GRID_N,),
        in_specs=in_specs,
        out_specs=[pl.BlockSpec((MB, HID), lambda i: (i, 0)),
                   pl.BlockSpec((MB, HID), lambda i: (i, 0))],
        out_shape=[jax.ShapeDtypeStruct((N, HID), jnp.float32),
                   jax.ShapeDtypeStruct((N, HID), jnp.float32)],
    )(h, su, vacc, dist, w, b)


def kernel(x, edge_index, edge_attr, ea_w1, ea_b1, ea_w2, ea_b2,
           tag1_w, tag1_b, tag2_w, tag2_b, tag3_w, tag3_b):
    xf = x[:, 4:4 + NFEAT]
    ei = edge_index.astype(jnp.int32)
    row0, col0 = ei[0], ei[1]
    directed = ~jnp.any((row0 == col0[0]) & (col0 == row0[0]))

    pad = EP - 2 * E
    arp = jnp.arange(pad, dtype=jnp.int32)
    zpad = arp % N
    ar = jnp.arange(E, dtype=jnp.int32)
    gcol = jnp.concatenate([col0, row0, zpad])
    grow = jnp.concatenate([row0, col0, zpad])
    ridx = jnp.concatenate([ar, ar, arp % E])
    dummy_rev = DUMMY + (ar % (NACC - N))
    dummy_pad = DUMMY + (arp % (NACC - N))
    scat = jnp.concatenate([col0, jnp.where(directed, row0, dummy_rev),
                            dummy_pad])
    packed = jnp.stack([gcol, grow, ridx, scat], axis=0)
    ipkm = packed.reshape(4, NCHUNKM, CM).transpose(1, 0, 2)
    ipk2 = packed.reshape(4, NCHUNK, C).transpose(1, 0, 2)

    w01 = jnp.concatenate([ea_w1[:NFEAT], ea_w1[NFEAT:2 * NFEAT]], axis=1)
    PQ = _pre1(xf, w01)
    P = PQ[:, :HID]
    Q = PQ[:, HID:]
    R = _pre2(edge_attr, ea_w1[2 * NFEAT:], ea_b1.reshape(1, HID))

    hacc = _msg_call(ipkm, P, Q, R)
    degp = _degs_call(ipk2)
    h, t, dist = _combine(hacc, degp, ea_w2, ea_b2.reshape(1, HID))

    for (w, b, final) in ((tag1_w, tag1_b, False),
                          (tag2_w, tag2_b, False),
                          (tag3_w, tag3_b, True)):
        uacc = _prop_call(ipk2, t)
        su, t2 = _mid(uacc, dist)
        vacc = _prop_call(ipk2, t2)
        res = _post(h, su, vacc, dist, w, b.reshape(1, HID), final)
        if final:
            return res
        h, t = res
